# initial kernel scaffold (unmeasured)
import jax
import jax.numpy as jnp
from jax import lax
from jax.experimental import pallas as pl
from jax.experimental.pallas import tpu as pltpu

N_DEV = 4
SQ = 256
SKV = 4096
HL = 8
DH = 128
DM = 1024
SCALE = 0.08838834764831843


def kernel(x, Wq, K_ext, V_ext, Wo):
    def body(x_ref, wq_ref, k_hbm, v_hbm, wo_ref, out_ref,
             kbuf, vbuf, comm, dma_sems, send_sems, recv_sems):
        p = lax.axis_index("i")
        right = lax.rem(p + 1, N_DEV)
        left = lax.rem(p + N_DEV - 1, N_DEV)

        kcopy = pltpu.make_async_copy(
            k_hbm.at[0, :, pl.ds(HL * p, HL), :], kbuf, dma_sems.at[0])
        vcopy = pltpu.make_async_copy(
            v_hbm.at[0, :, pl.ds(HL * p, HL), :], vbuf, dma_sems.at[1])
        kcopy.start()
        vcopy.start()

        barrier_sem = pltpu.get_barrier_semaphore()
        for nbr in [left, right]:
            pl.semaphore_signal(
                barrier_sem, inc=1,
                device_id=(nbr,), device_id_type=pl.DeviceIdType.MESH)
        pl.semaphore_wait(barrier_sem, 2)

        q = jnp.dot(x_ref[0], wq_ref[:, :], preferred_element_type=jnp.float32)

        ri = lax.broadcasted_iota(jnp.int32, (SQ, SKV), 0)
        ci = lax.broadcasted_iota(jnp.int32, (SQ, SKV), 1)
        mask = ((ci // 64) % 4) == (ri // 64)

        kcopy.wait()
        vcopy.wait()

        ctx_parts = []
        for h in range(HL):
            qh = q[:, h * DH:(h + 1) * DH]
            kh = kbuf[:, h, :]
            vh = vbuf[:, h, :]
            s = lax.dot_general(
                qh, kh, (((1,), (1,)), ((), ())),
                preferred_element_type=jnp.float32) * SCALE
            s = jnp.where(mask, s, -1e9)
            m = jnp.max(s, axis=1, keepdims=True)
            w = jnp.exp(s - m)
            d = jnp.sum(w, axis=1, keepdims=True)
            w = w / d
            ctx_parts.append(
                jnp.dot(w, vh, preferred_element_type=jnp.float32))
        ctx = jnp.concatenate(ctx_parts, axis=1)
        partial = jnp.dot(ctx, wo_ref[:, :], preferred_element_type=jnp.float32)

        out_ref[0] = partial
        comm[0] = partial
        for hop in range(N_DEV - 1):
            send_slot = hop % 2
            recv_slot = (hop + 1) % 2
            rdma = pltpu.make_async_remote_copy(
                src_ref=comm.at[send_slot],
                dst_ref=comm.at[recv_slot],
                send_sem=send_sems.at[send_slot],
                recv_sem=recv_sems.at[recv_slot],
                device_id=(right,),
                device_id_type=pl.DeviceIdType.MESH)
            rdma.start()
            rdma.wait()
            out_ref[0] = out_ref[0] + comm[recv_slot]

    return pl.pallas_call(
        body,
        out_shape=jax.ShapeDtypeStruct((1, SQ, DM), jnp.float32),
        in_specs=[
            pl.BlockSpec(memory_space=pltpu.VMEM),
            pl.BlockSpec(memory_space=pltpu.VMEM),
            pl.BlockSpec(memory_space=pltpu.ANY),
            pl.BlockSpec(memory_space=pltpu.ANY),
            pl.BlockSpec(memory_space=pltpu.VMEM),
        ],
        out_specs=pl.BlockSpec(memory_space=pltpu.VMEM),
        scratch_shapes=[
            pltpu.VMEM((SKV, HL, DH), jnp.float32),
            pltpu.VMEM((SKV, HL, DH), jnp.float32),
            pltpu.VMEM((2, SQ, DM), jnp.float32),
            pltpu.SemaphoreType.DMA((2,)),
            pltpu.SemaphoreType.DMA((2,)),
            pltpu.SemaphoreType.DMA((2,)),
        ],
        compiler_params=pltpu.CompilerParams(collective_id=0),
    )(x, Wq, K_ext, V_ext, Wo)


# baseline (device time: 94459 ns/iter reference)
import jax
import jax.numpy as jnp
from jax import lax
from jax.experimental import pallas as pl
from jax.experimental.pallas import tpu as pltpu

N_DEV = 4
SQ = 256
SKV = 4096
HL = 8
DH = 128
DM = 1024
SCALE = 0.08838834764831843


def kernel(x, Wq, K_ext, V_ext, Wo):
    def body(x_ref, wq_ref, k_hbm, v_hbm, wo_ref, out_ref,
             kbuf, vbuf, comm, dma_sems, send_sems, recv_sems):
        p = lax.axis_index("i")
        right = lax.rem(p + 1, N_DEV)
        left = lax.rem(p + N_DEV - 1, N_DEV)

        kcopy = pltpu.make_async_copy(
            k_hbm.at[0, :, pl.ds(HL * p, HL), :], kbuf, dma_sems.at[0])
        vcopy = pltpu.make_async_copy(
            v_hbm.at[0, :, pl.ds(HL * p, HL), :], vbuf, dma_sems.at[1])
        kcopy.start()
        vcopy.start()

        barrier_sem = pltpu.get_barrier_semaphore()
        for nbr in [left, right]:
            pl.semaphore_signal(
                barrier_sem, inc=1,
                device_id=(nbr,), device_id_type=pl.DeviceIdType.MESH)
        pl.semaphore_wait(barrier_sem, 2)

        q = jnp.dot(x_ref[0], wq_ref[:, :], preferred_element_type=jnp.float32)

        ri = lax.broadcasted_iota(jnp.int32, (SQ, SKV), 0)
        ci = lax.broadcasted_iota(jnp.int32, (SQ, SKV), 1)
        mask = ((ci // 64) % 4) == (ri // 64)

        kcopy.wait()
        vcopy.wait()

        ctx_parts = []
        for h in range(HL):
            qh = q[:, h * DH:(h + 1) * DH]
            kh = kbuf[:, h, :]
            vh = vbuf[:, h, :]
            s = lax.dot_general(
                qh, kh, (((1,), (1,)), ((), ())),
                preferred_element_type=jnp.float32) * SCALE
            s = jnp.where(mask, s, -1e9)
            m = jnp.max(s, axis=1, keepdims=True)
            w = jnp.exp(s - m)
            d = jnp.sum(w, axis=1, keepdims=True)
            w = w / d
            ctx_parts.append(
                jnp.dot(w, vh, preferred_element_type=jnp.float32))
        ctx = jnp.concatenate(ctx_parts, axis=1)
        partial = jnp.dot(ctx, wo_ref[:, :], preferred_element_type=jnp.float32)

        out_ref[0] = partial
        comm[0] = partial
        for hop in range(N_DEV - 1):
            rdma = pltpu.make_async_remote_copy(
                src_ref=comm.at[hop],
                dst_ref=comm.at[hop + 1],
                send_sem=send_sems.at[hop],
                recv_sem=recv_sems.at[hop],
                device_id=(right,),
                device_id_type=pl.DeviceIdType.MESH)
            rdma.start()
            rdma.wait()
            out_ref[0] = out_ref[0] + comm[hop + 1]

    return pl.pallas_call(
        body,
        out_shape=jax.ShapeDtypeStruct((1, SQ, DM), jnp.float32),
        in_specs=[
            pl.BlockSpec(memory_space=pltpu.VMEM),
            pl.BlockSpec(memory_space=pltpu.VMEM),
            pl.BlockSpec(memory_space=pl.ANY),
            pl.BlockSpec(memory_space=pl.ANY),
            pl.BlockSpec(memory_space=pltpu.VMEM),
        ],
        out_specs=pl.BlockSpec(memory_space=pltpu.VMEM),
        scratch_shapes=[
            pltpu.VMEM((SKV, HL, DH), jnp.float32),
            pltpu.VMEM((SKV, HL, DH), jnp.float32),
            pltpu.VMEM((N_DEV, SQ, DM), jnp.float32),
            pltpu.SemaphoreType.DMA((2,)),
            pltpu.SemaphoreType.DMA((N_DEV - 1,)),
            pltpu.SemaphoreType.DMA((N_DEV - 1,)),
        ],
        compiler_params=pltpu.CompilerParams(
            collective_id=0,
            vmem_limit_bytes=60 * 1024 * 1024,
        ),
    )(x, Wq, K_ext, V_ext, Wo)


# device time: 79522 ns/iter; 1.1878x vs baseline; 1.1878x over previous
import jax
import jax.numpy as jnp
from jax import lax
from jax.experimental import pallas as pl
from jax.experimental.pallas import tpu as pltpu

N_DEV = 4
SQ = 256
SKV = 4096
HL = 8
DH = 128
DM = 1024
QB = 64
NC = 4
KPC = SKV // NC
NSB = KPC // QB
SCALE = 0.08838834764831843


def kernel(x, Wq, K_ext, V_ext, Wo):
    def body(x_ref, wq_ref, k_hbm, v_hbm, wo_ref, out_ref,
             kbuf, vbuf, comm, kv_sems, send_sems, recv_sems):
        p = lax.axis_index("i")
        right = lax.rem(p + 1, N_DEV)
        left = lax.rem(p + N_DEV - 1, N_DEV)

        def kv_copies():
            copies = []
            for c in range(NC):
                for sb in range(NSB):
                    row0 = QB * (NC * sb + c)
                    copies.append(pltpu.make_async_copy(
                        k_hbm.at[0, pl.ds(row0, QB), pl.ds(HL * p, HL), :],
                        kbuf.at[c, pl.ds(QB * sb, QB), :, :],
                        kv_sems.at[0, c]))
                    copies.append(pltpu.make_async_copy(
                        v_hbm.at[0, pl.ds(row0, QB), pl.ds(HL * p, HL), :],
                        vbuf.at[c, pl.ds(QB * sb, QB), :, :],
                        kv_sems.at[1, c]))
            return copies

        for cp in kv_copies():
            cp.start()

        barrier_sem = pltpu.get_barrier_semaphore()
        for nbr in [left, right]:
            pl.semaphore_signal(
                barrier_sem, inc=1,
                device_id=(nbr,), device_id_type=pl.DeviceIdType.MESH)
        pl.semaphore_wait(barrier_sem, 2)

        q = jnp.dot(x_ref[0], wq_ref[:, :], preferred_element_type=jnp.float32)

        waiters = kv_copies()
        for c in range(NC):
            for cp in waiters[2 * NSB * c:2 * NSB * (c + 1)]:
                cp.wait()
            qc = q[QB * c:QB * (c + 1), :]
            ctx_parts = []
            for h in range(HL):
                qh = qc[:, h * DH:(h + 1) * DH]
                kh = kbuf[c, :, h, :]
                vh = vbuf[c, :, h, :]
                s = lax.dot_general(
                    qh, kh, (((1,), (1,)), ((), ())),
                    preferred_element_type=jnp.float32) * SCALE
                m = jnp.max(s, axis=1, keepdims=True)
                w = jnp.exp(s - m)
                d = jnp.sum(w, axis=1, keepdims=True)
                w = w / d
                ctx_parts.append(
                    jnp.dot(w, vh, preferred_element_type=jnp.float32))
            ctx_c = jnp.concatenate(ctx_parts, axis=1)
            comm[0, pl.ds(QB * c, QB), :] = jnp.dot(
                ctx_c, wo_ref[:, :], preferred_element_type=jnp.float32)

        out_ref[0] = comm[0]
        for hop in range(N_DEV - 1):
            rdma = pltpu.make_async_remote_copy(
                src_ref=comm.at[hop],
                dst_ref=comm.at[hop + 1],
                send_sem=send_sems.at[hop],
                recv_sem=recv_sems.at[hop],
                device_id=(right,),
                device_id_type=pl.DeviceIdType.MESH)
            rdma.start()
            rdma.wait()
            out_ref[0] = out_ref[0] + comm[hop + 1]

    return pl.pallas_call(
        body,
        out_shape=jax.ShapeDtypeStruct((1, SQ, DM), jnp.float32),
        in_specs=[
            pl.BlockSpec(memory_space=pltpu.VMEM),
            pl.BlockSpec(memory_space=pltpu.VMEM),
            pl.BlockSpec(memory_space=pl.ANY),
            pl.BlockSpec(memory_space=pl.ANY),
            pl.BlockSpec(memory_space=pltpu.VMEM),
        ],
        out_specs=pl.BlockSpec(memory_space=pltpu.VMEM),
        scratch_shapes=[
            pltpu.VMEM((NC, KPC, HL, DH), jnp.float32),
            pltpu.VMEM((NC, KPC, HL, DH), jnp.float32),
            pltpu.VMEM((N_DEV, SQ, DM), jnp.float32),
            pltpu.SemaphoreType.DMA((2, NC)),
            pltpu.SemaphoreType.DMA((N_DEV - 1,)),
            pltpu.SemaphoreType.DMA((N_DEV - 1,)),
        ],
        compiler_params=pltpu.CompilerParams(
            collective_id=0,
            vmem_limit_bytes=60 * 1024 * 1024,
        ),
    )(x, Wq, K_ext, V_ext, Wo)


# device time: 62872 ns/iter; 1.5024x vs baseline; 1.2648x over previous
import jax
import jax.numpy as jnp
from jax import lax
from jax.experimental import pallas as pl
from jax.experimental.pallas import tpu as pltpu

N_DEV = 4
SQ = 256
SKV = 4096
HL = 8
DH = 128
DM = 1024
QB = 64
NC = 4
KPC = SKV // NC
NSB = KPC // QB
SCALE = 0.08838834764831843


def kernel(x, Wq, K_ext, V_ext, Wo):
    def body(x_ref, wq_ref, k_hbm, v_hbm, wo_ref, out_ref,
             kbuf, vbuf, comm, kv_sems, send_sems, recv_sems):
        p = lax.axis_index("i")
        right = lax.rem(p + 1, N_DEV)
        left = lax.rem(p + N_DEV - 1, N_DEV)

        def kv_copies():
            copies = []
            for c in range(NC):
                for sb in range(NSB):
                    row0 = QB * (NC * sb + c)
                    copies.append(pltpu.make_async_copy(
                        k_hbm.at[0, pl.ds(row0, QB), pl.ds(HL * p, HL), :],
                        kbuf.at[c, pl.ds(QB * sb, QB), :, :],
                        kv_sems.at[0, c]))
                    copies.append(pltpu.make_async_copy(
                        v_hbm.at[0, pl.ds(row0, QB), pl.ds(HL * p, HL), :],
                        vbuf.at[c, pl.ds(QB * sb, QB), :, :],
                        kv_sems.at[1, c]))
            return copies

        for cp in kv_copies():
            cp.start()

        barrier_sem = pltpu.get_barrier_semaphore()
        for nbr in [left, right]:
            pl.semaphore_signal(
                barrier_sem, inc=1,
                device_id=(nbr,), device_id_type=pl.DeviceIdType.MESH)
        pl.semaphore_wait(barrier_sem, 2)

        q = jnp.dot(x_ref[0], wq_ref[:, :], preferred_element_type=jnp.float32)

        rdmas = {}

        def service(c, h):
            if h > 0:
                rdmas[(c, h - 1)].wait_recv()
            r = pltpu.make_async_remote_copy(
                src_ref=comm.at[c, h],
                dst_ref=comm.at[c, h + 1],
                send_sem=send_sems.at[c, h],
                recv_sem=recv_sems.at[c, h],
                device_id=(right,),
                device_id_type=pl.DeviceIdType.MESH)
            rdmas[(c, h)] = r
            r.start()

        waiters = kv_copies()
        for c in range(NC):
            for cp in waiters[2 * NSB * c:2 * NSB * (c + 1)]:
                cp.wait()
            qc = q[QB * c:QB * (c + 1), :]
            ctx_parts = []
            for h in range(HL):
                qh = qc[:, h * DH:(h + 1) * DH]
                kh = kbuf[c, :, h, :]
                vh = vbuf[c, :, h, :]
                s = lax.dot_general(
                    qh, kh, (((1,), (1,)), ((), ())),
                    preferred_element_type=jnp.float32) * SCALE
                m = jnp.max(s, axis=1, keepdims=True)
                w = jnp.exp(s - m)
                d = jnp.sum(w, axis=1, keepdims=True)
                w = w / d
                ctx_parts.append(
                    jnp.dot(w, vh, preferred_element_type=jnp.float32))
            ctx_c = jnp.concatenate(ctx_parts, axis=1)
            comm[c, 0] = jnp.dot(
                ctx_c, wo_ref[:, :], preferred_element_type=jnp.float32)
            service(c, 0)
            if c >= 1:
                service(c - 1, 1)
            if c >= 2:
                service(c - 2, 2)

        service(2, 2)
        service(3, 1)
        service(3, 2)
        for c in range(NC):
            rdmas[(c, N_DEV - 2)].wait_recv()
            out_ref[0, QB * c:QB * (c + 1), :] = (
                (comm[c, 0] + comm[c, 1]) + (comm[c, 2] + comm[c, 3]))
        for r in rdmas.values():
            r.wait_send()

    return pl.pallas_call(
        body,
        out_shape=jax.ShapeDtypeStruct((1, SQ, DM), jnp.float32),
        in_specs=[
            pl.BlockSpec(memory_space=pltpu.VMEM),
            pl.BlockSpec(memory_space=pltpu.VMEM),
            pl.BlockSpec(memory_space=pl.ANY),
            pl.BlockSpec(memory_space=pl.ANY),
            pl.BlockSpec(memory_space=pltpu.VMEM),
        ],
        out_specs=pl.BlockSpec(memory_space=pltpu.VMEM),
        scratch_shapes=[
            pltpu.VMEM((NC, KPC, HL, DH), jnp.float32),
            pltpu.VMEM((NC, KPC, HL, DH), jnp.float32),
            pltpu.VMEM((NC, N_DEV, QB, DM), jnp.float32),
            pltpu.SemaphoreType.DMA((2, NC)),
            pltpu.SemaphoreType.DMA((NC, N_DEV - 1)),
            pltpu.SemaphoreType.DMA((NC, N_DEV - 1)),
        ],
        compiler_params=pltpu.CompilerParams(
            collective_id=0,
            vmem_limit_bytes=60 * 1024 * 1024,
        ),
    )(x, Wq, K_ext, V_ext, Wo)
